# baseline (speedup 1.0000x reference)
import jax
import jax.numpy as jnp
from jax import lax
from jax.experimental import pallas as pl
from jax.experimental.pallas import tpu as pltpu
from jax.experimental.pallas import tpu_sc as plsc

B, A, F, D = 64, 512, 128, 16
LANES = 16
W = F // 2
WPAD = W + 1
NGROUPS = W // LANES

NC, NS = 2, 16
NW = NC * NS
MOLS_PER_W = B // NW
ACHUNK = 128
NACH = A // ACHUNK


def _dyn_gather(vec, idx):
    dn = lax.GatherDimensionNumbers(
        offset_dims=(), collapsed_slice_dims=(0,), start_index_map=(0,))
    return lax.gather(vec, idx[:, None], dn, (1,),
                      mode=lax.GatherScatterMode.PROMISE_IN_BOUNDS)


def _graph_pool_body(atoms_hbm, edges_hbm, out_hbm, atoms_v, edges_v, out_v, sem):
    wid = lax.axis_index("s") * NC + lax.axis_index("c")

    lanes = lax.broadcasted_iota(jnp.int32, (LANES,), 0)
    gbases = [lanes + g * LANES for g in range(NGROUPS)]
    colA = [lanes + g * LANES for g in range(NGROUPS)]
    colB = [lanes + W + g * LANES for g in range(NGROUPS)]
    zerov = jnp.zeros((LANES,), jnp.int32)
    rotv = (lanes + 1) & (LANES - 1)

    for m in range(MOLS_PER_W):
        b = wid * MOLS_PER_W + m
        pltpu.sync_copy(edges_hbm.at[pl.ds(b * A, A)], edges_v)
        pltpu.sync_copy(atoms_hbm.at[pl.ds(b * A, A)], atoms_v)

        for ch in range(NACH):
            def atom_body(a, ch=ch):
                aa = ch * ACHUNK + a
                selfv = jnp.full((LANES,), aa, jnp.int32)
                av = jnp.full((LANES,), a, jnp.int32)
                ev = plsc.load_gather(edges_v, [selfv, lanes])
                accs = [
                    plsc.bitcast(
                        plsc.load_gather(atoms_v, [selfv, gbases[g]]),
                        jnp.bfloat16)
                    for g in range(NGROUPS)]
                for d in range(D):
                    rowv = _dyn_gather(ev, zerov)
                    if d + 1 < D:
                        ev = _dyn_gather(ev, rotv)
                    for g in range(NGROUPS):
                        w = plsc.load_gather(atoms_v, [rowv, gbases[g]])
                        accs[g] = jnp.maximum(
                            accs[g], plsc.bitcast(w, jnp.bfloat16))
                for g in range(NGROUPS):
                    evens, odds = plsc.unpack(
                        accs[g], format=plsc.PackFormat.INTERLEAVED)
                    plsc.store_scatter(out_v, [av, colA[g]], evens)
                    plsc.store_scatter(out_v, [av, colB[g]], odds)

            plsc.parallel_loop(0, ACHUNK)(atom_body)
            pltpu.sync_copy(
                out_v, out_hbm.at[pl.ds(b * A + ch * ACHUNK, ACHUNK)])


_graph_pool = pl.kernel(
    _graph_pool_body,
    out_type=jax.ShapeDtypeStruct((B * A, F), jnp.float32),
    mesh=plsc.VectorSubcoreMesh(core_axis_name="c", subcore_axis_name="s"),
    scratch_types=[
        pltpu.VMEM((A, WPAD), jnp.int32),
        pltpu.VMEM((A, D), jnp.int32),
        pltpu.VMEM((ACHUNK, F), jnp.float32),
        pltpu.SemaphoreType.DMA,
    ],
    compiler_params=pltpu.CompilerParams(
        use_tc_tiling_on_sc=False, needs_layout_passes=False),
)


def kernel(atoms, edges):
    bits = jax.lax.bitcast_convert_type(atoms.reshape(B * A, F), jnp.int32)
    r = (bits + 0x7FFF + ((bits >> 16) & 1)) >> 16
    lo, hi = r[:, :W], r[:, W:]
    packed = (hi << 16) | (lo & 0xFFFF)
    packed = jnp.concatenate(
        [packed, jnp.zeros((B * A, WPAD - W), jnp.int32)], axis=-1)
    out = _graph_pool(packed, edges.astype(jnp.int32).reshape(B * A, D))
    return out.reshape(B, A, F)


# speedup vs baseline: 1.4189x; 1.4189x over previous
import jax
import jax.numpy as jnp
from jax import lax
from jax.experimental import pallas as pl
from jax.experimental.pallas import tpu as pltpu
from jax.experimental.pallas import tpu_sc as plsc

B, A, F, D = 64, 512, 128, 16
LANES = 16
W = F // 2
WPAD = W + 1
NGROUPS = W // LANES

NC, NS = 2, 16
NW = NC * NS
MOLS_PER_W = B // NW
ACHUNK = 128
NACH = A // ACHUNK


def _dyn_gather(vec, idx):
    dn = lax.GatherDimensionNumbers(
        offset_dims=(), collapsed_slice_dims=(0,), start_index_map=(0,))
    return lax.gather(vec, idx[:, None], dn, (1,),
                      mode=lax.GatherScatterMode.PROMISE_IN_BOUNDS)


def _graph_pool_body(atoms_hbm, edges_hbm, out_hbm,
                     stage_v, atoms_v, edges_v, out_v, sem):
    wid = lax.axis_index("s") * NC + lax.axis_index("c")

    lanes = lax.broadcasted_iota(jnp.int32, (LANES,), 0)
    gbases = [lanes + g * LANES for g in range(NGROUPS)]
    colA = [lanes + g * 2 * LANES for g in range(NGROUPS)]
    colB = [lanes + g * 2 * LANES + LANES for g in range(NGROUPS)]
    zerov = jnp.zeros((LANES,), jnp.int32)
    rotv = (lanes + 1) & (LANES - 1)

    for m in range(MOLS_PER_W):
        b = wid * MOLS_PER_W + m
        pltpu.sync_copy(edges_hbm.at[pl.ds(b * D, D)], edges_v)

        for ch in range(NACH):
            pltpu.sync_copy(
                atoms_hbm.at[pl.ds(b * A + ch * ACHUNK, ACHUNK)], stage_v)

            def pack_row(r, ch=ch):
                rv = jnp.full((LANES,), r, jnp.int32)
                prv = jnp.full((LANES,), ch * ACHUNK + r, jnp.int32)
                for g in range(NGROUPS):
                    a = plsc.load_gather(stage_v, [rv, colA[g]])
                    o = plsc.load_gather(stage_v, [rv, colB[g]])
                    w = plsc.bitcast(
                        plsc.pack(a, o, format=plsc.PackFormat.INTERLEAVED),
                        jnp.int32)
                    plsc.store_scatter(atoms_v, [prv, gbases[g]], w)

            plsc.parallel_loop(0, ACHUNK)(pack_row)

        for ch in range(NACH):
            def atom_body(a, ch=ch):
                aa = ch * ACHUNK + a
                selfv = jnp.full((LANES,), aa, jnp.int32)
                av = jnp.full((LANES,), a, jnp.int32)
                ev = plsc.load_gather(edges_v, [lanes, selfv])
                accs = [
                    plsc.bitcast(
                        plsc.load_gather(atoms_v, [selfv, gbases[g]]),
                        jnp.bfloat16)
                    for g in range(NGROUPS)]
                for d in range(D):
                    rowv = _dyn_gather(ev, zerov)
                    if d + 1 < D:
                        ev = _dyn_gather(ev, rotv)
                    for g in range(NGROUPS):
                        w = plsc.load_gather(atoms_v, [rowv, gbases[g]])
                        accs[g] = jnp.maximum(
                            accs[g], plsc.bitcast(w, jnp.bfloat16))
                for g in range(NGROUPS):
                    evens, odds = plsc.unpack(
                        accs[g], format=plsc.PackFormat.INTERLEAVED)
                    plsc.store_scatter(out_v, [av, colA[g]], evens)
                    plsc.store_scatter(out_v, [av, colB[g]], odds)

            plsc.parallel_loop(0, ACHUNK)(atom_body)
            pltpu.sync_copy(
                out_v, out_hbm.at[pl.ds(b * A + ch * ACHUNK, ACHUNK)])


_graph_pool = pl.kernel(
    _graph_pool_body,
    out_type=jax.ShapeDtypeStruct((B * A, F), jnp.float32),
    mesh=plsc.VectorSubcoreMesh(core_axis_name="c", subcore_axis_name="s"),
    scratch_types=[
        pltpu.VMEM((ACHUNK, F), jnp.float32),
        pltpu.VMEM((A, WPAD), jnp.int32),
        pltpu.VMEM((D, A), jnp.int32),
        pltpu.VMEM((ACHUNK, F), jnp.float32),
        pltpu.SemaphoreType.DMA,
    ],
    compiler_params=pltpu.CompilerParams(
        use_tc_tiling_on_sc=False, needs_layout_passes=False),
)


def kernel(atoms, edges):
    edges_t = edges.astype(jnp.int32).transpose(0, 2, 1).reshape(B * D, A)
    out = _graph_pool(atoms.reshape(B * A, F), edges_t)
    return out.reshape(B, A, F)


# speedup vs baseline: 1.4867x; 1.0478x over previous
import jax
import jax.numpy as jnp
from jax import lax
from jax.experimental import pallas as pl
from jax.experimental.pallas import tpu as pltpu
from jax.experimental.pallas import tpu_sc as plsc

B, A, F, D = 64, 512, 128, 16
LANES = 16
W = F // 2
WPAD = W + 1
NGROUPS = W // LANES

NC, NS = 2, 16
NW = NC * NS
MOLS_PER_W = B // NW
ACHUNK = 128
NACH = A // ACHUNK


def _dyn_gather(vec, idx):
    dn = lax.GatherDimensionNumbers(
        offset_dims=(), collapsed_slice_dims=(0,), start_index_map=(0,))
    return lax.gather(vec, idx[:, None], dn, (1,),
                      mode=lax.GatherScatterMode.PROMISE_IN_BOUNDS)


def _graph_pool_body(atoms_hbm, edges_hbm, out_hbm,
                     stage_v, atoms_v, edges_v, out_v0, out_v1, sem0, sem1):
    wid = lax.axis_index("s") * NC + lax.axis_index("c")
    outs, sems = [out_v0, out_v1], [sem0, sem1]
    pending = [None, None]

    lanes = lax.broadcasted_iota(jnp.int32, (LANES,), 0)
    gbases = [lanes + g * LANES for g in range(NGROUPS)]
    colA = [lanes + g * 2 * LANES for g in range(NGROUPS)]
    colB = [lanes + g * 2 * LANES + LANES for g in range(NGROUPS)]
    zerov = jnp.zeros((LANES,), jnp.int32)
    rotv = (lanes + 1) & (LANES - 1)

    for m in range(MOLS_PER_W):
        b = wid * MOLS_PER_W + m
        pltpu.sync_copy(edges_hbm.at[pl.ds(b * D, D)], edges_v)

        for ch in range(NACH):
            pltpu.sync_copy(
                atoms_hbm.at[pl.ds(b * A + ch * ACHUNK, ACHUNK)], stage_v)

            def pack_row(r, ch=ch):
                rv = jnp.full((LANES,), r, jnp.int32)
                prv = jnp.full((LANES,), ch * ACHUNK + r, jnp.int32)
                for g in range(NGROUPS):
                    a = plsc.load_gather(stage_v, [rv, colA[g]])
                    o = plsc.load_gather(stage_v, [rv, colB[g]])
                    w = plsc.bitcast(
                        plsc.pack(a, o, format=plsc.PackFormat.INTERLEAVED),
                        jnp.int32)
                    plsc.store_scatter(atoms_v, [prv, gbases[g]], w)

            plsc.parallel_loop(0, ACHUNK)(pack_row)

        for ch in range(NACH):
            k = (m * NACH + ch) % 2
            out_v = outs[k]
            if pending[k] is not None:
                pending[k].wait()
                pending[k] = None

            def atom_body(a, ch=ch, out_v=out_v):
                aa = ch * ACHUNK + a
                selfv = jnp.full((LANES,), aa, jnp.int32)
                av = jnp.full((LANES,), a, jnp.int32)
                ev = plsc.load_gather(edges_v, [lanes, selfv])
                accs = [
                    plsc.bitcast(
                        plsc.load_gather(atoms_v, [selfv, gbases[g]]),
                        jnp.bfloat16)
                    for g in range(NGROUPS)]
                for d in range(D):
                    rowv = _dyn_gather(ev, zerov)
                    if d + 1 < D:
                        ev = _dyn_gather(ev, rotv)
                    for g in range(NGROUPS):
                        w = plsc.load_gather(atoms_v, [rowv, gbases[g]])
                        accs[g] = jnp.maximum(
                            accs[g], plsc.bitcast(w, jnp.bfloat16))
                for g in range(NGROUPS):
                    evens, odds = plsc.unpack(
                        accs[g], format=plsc.PackFormat.INTERLEAVED)
                    plsc.store_scatter(out_v, [av, colA[g]], evens)
                    plsc.store_scatter(out_v, [av, colB[g]], odds)

            plsc.parallel_loop(0, ACHUNK)(atom_body)
            pending[k] = pltpu.async_copy(
                out_v, out_hbm.at[pl.ds(b * A + ch * ACHUNK, ACHUNK)], sems[k])

    for k in (0, 1):
        if pending[k] is not None:
            pending[k].wait()


_graph_pool = pl.kernel(
    _graph_pool_body,
    out_type=jax.ShapeDtypeStruct((B * A, F), jnp.float32),
    mesh=plsc.VectorSubcoreMesh(core_axis_name="c", subcore_axis_name="s"),
    scratch_types=[
        pltpu.VMEM((ACHUNK, F), jnp.float32),
        pltpu.VMEM((A, WPAD), jnp.int32),
        pltpu.VMEM((D, A), jnp.int32),
        pltpu.VMEM((ACHUNK, F), jnp.float32),
        pltpu.VMEM((ACHUNK, F), jnp.float32),
        pltpu.SemaphoreType.DMA,
        pltpu.SemaphoreType.DMA,
    ],
    compiler_params=pltpu.CompilerParams(
        use_tc_tiling_on_sc=False, needs_layout_passes=False),
)


def kernel(atoms, edges):
    edges_t = edges.astype(jnp.int32).transpose(0, 2, 1).reshape(B * D, A)
    out = _graph_pool(atoms.reshape(B * A, F), edges_t)
    return out.reshape(B, A, F)


# speedup vs baseline: 1.5620x; 1.0507x over previous
import jax
import jax.numpy as jnp
from jax import lax
from jax.experimental import pallas as pl
from jax.experimental.pallas import tpu as pltpu
from jax.experimental.pallas import tpu_sc as plsc

B, A, F, D = 64, 512, 128, 16
LANES = 16
W = F // 2
WPAD = W + 1
NGROUPS = W // LANES

NC, NS = 2, 16
NW = NC * NS
MOLS_PER_W = B // NW
ACHUNK = 128
NACH = A // ACHUNK


def _dyn_gather(vec, idx):
    dn = lax.GatherDimensionNumbers(
        offset_dims=(), collapsed_slice_dims=(0,), start_index_map=(0,))
    return lax.gather(vec, idx[:, None], dn, (1,),
                      mode=lax.GatherScatterMode.PROMISE_IN_BOUNDS)


def _graph_pool_body(atoms_hbm, edges_hbm, out_hbm,
                     stage_v0, stage_v1, atoms_v, edges_v,
                     out_v0, out_v1, esem, ssem0, ssem1, sem0, sem1):
    wid = lax.axis_index("s") * NC + lax.axis_index("c")
    outs, sems = [out_v0, out_v1], [sem0, sem1]
    stages, ssems = [stage_v0, stage_v1], [ssem0, ssem1]
    pending = [None, None]

    lanes = lax.broadcasted_iota(jnp.int32, (LANES,), 0)
    gbases = [lanes + g * LANES for g in range(NGROUPS)]
    colA = [lanes + g * 2 * LANES for g in range(NGROUPS)]
    colB = [lanes + g * 2 * LANES + LANES for g in range(NGROUPS)]
    zerov = jnp.zeros((LANES,), jnp.int32)
    rotv = (lanes + 1) & (LANES - 1)

    for m in range(MOLS_PER_W):
        b = wid * MOLS_PER_W + m
        eh = pltpu.async_copy(edges_hbm.at[pl.ds(b * D, D)], edges_v, esem)

        sh = pltpu.async_copy(
            atoms_hbm.at[pl.ds(b * A, ACHUNK)], stages[0], ssems[0])
        for ch in range(NACH):
            stage_v = stages[ch % 2]
            sh.wait()
            if ch + 1 < NACH:
                sh = pltpu.async_copy(
                    atoms_hbm.at[pl.ds(b * A + (ch + 1) * ACHUNK, ACHUNK)],
                    stages[(ch + 1) % 2], ssems[(ch + 1) % 2])

            def pack_row(r, ch=ch, stage_v=stage_v):
                rv = jnp.full((LANES,), r, jnp.int32)
                prv = jnp.full((LANES,), ch * ACHUNK + r, jnp.int32)
                for g in range(NGROUPS):
                    a = plsc.load_gather(stage_v, [rv, colA[g]])
                    o = plsc.load_gather(stage_v, [rv, colB[g]])
                    w = plsc.bitcast(
                        plsc.pack(a, o, format=plsc.PackFormat.INTERLEAVED),
                        jnp.int32)
                    plsc.store_scatter(atoms_v, [prv, gbases[g]], w)

            plsc.parallel_loop(0, ACHUNK)(pack_row)
        eh.wait()

        for ch in range(NACH):
            k = (m * NACH + ch) % 2
            out_v = outs[k]
            if pending[k] is not None:
                pending[k].wait()
                pending[k] = None

            def atom_body(a, ch=ch, out_v=out_v):
                aa = ch * ACHUNK + a
                selfv = jnp.full((LANES,), aa, jnp.int32)
                av = jnp.full((LANES,), a, jnp.int32)
                ev = plsc.load_gather(edges_v, [lanes, selfv])
                accs = [
                    plsc.bitcast(
                        plsc.load_gather(atoms_v, [selfv, gbases[g]]),
                        jnp.bfloat16)
                    for g in range(NGROUPS)]
                for d in range(D):
                    rowv = _dyn_gather(ev, zerov)
                    if d + 1 < D:
                        ev = _dyn_gather(ev, rotv)
                    for g in range(NGROUPS):
                        w = plsc.load_gather(atoms_v, [rowv, gbases[g]])
                        accs[g] = jnp.maximum(
                            accs[g], plsc.bitcast(w, jnp.bfloat16))
                for g in range(NGROUPS):
                    evens, odds = plsc.unpack(
                        accs[g], format=plsc.PackFormat.INTERLEAVED)
                    plsc.store_scatter(out_v, [av, colA[g]], evens)
                    plsc.store_scatter(out_v, [av, colB[g]], odds)

            plsc.parallel_loop(0, ACHUNK)(atom_body)
            pending[k] = pltpu.async_copy(
                out_v, out_hbm.at[pl.ds(b * A + ch * ACHUNK, ACHUNK)], sems[k])

    for k in (0, 1):
        if pending[k] is not None:
            pending[k].wait()


_graph_pool = pl.kernel(
    _graph_pool_body,
    out_type=jax.ShapeDtypeStruct((B * A, F), jnp.float32),
    mesh=plsc.VectorSubcoreMesh(core_axis_name="c", subcore_axis_name="s"),
    scratch_types=[
        pltpu.VMEM((ACHUNK, F), jnp.float32),
        pltpu.VMEM((ACHUNK, F), jnp.float32),
        pltpu.VMEM((A, WPAD), jnp.int32),
        pltpu.VMEM((D, A), jnp.int32),
        pltpu.VMEM((ACHUNK, F), jnp.float32),
        pltpu.VMEM((ACHUNK, F), jnp.float32),
        pltpu.SemaphoreType.DMA,
        pltpu.SemaphoreType.DMA,
        pltpu.SemaphoreType.DMA,
        pltpu.SemaphoreType.DMA,
        pltpu.SemaphoreType.DMA,
    ],
    compiler_params=pltpu.CompilerParams(
        use_tc_tiling_on_sc=False, needs_layout_passes=False),
)


def kernel(atoms, edges):
    edges_t = edges.astype(jnp.int32).transpose(0, 2, 1).reshape(B * D, A)
    out = _graph_pool(atoms.reshape(B * A, F), edges_t)
    return out.reshape(B, A, F)
